# baseline (speedup 1.0000x reference)
import functools

import jax
import jax.numpy as jnp
from jax import lax
from jax.experimental import pallas as pl
from jax.experimental.pallas import tpu as pltpu
from jax.experimental.pallas import tpu_sc as plsc

B, L = 4096, 50
N = B * L
D = 32
OUTD = 3 * D
NW = 32
BB = B // NW
NP = L // 2
GROUPS = (2 * BB) // 16


def _make_kernel():
    info = plsc.get_sparse_core_info()
    nc = info.num_cores
    mesh = plsc.VectorSubcoreMesh(core_axis_name="c", subcore_axis_name="s")

    @functools.partial(
        pl.kernel,
        mesh=mesh,
        out_type=jax.ShapeDtypeStruct((L, OUTD // 8, NW, 8, BB), jnp.float32),
        compiler_params=pltpu.CompilerParams(use_tc_tiling_on_sc=False,
                                             needs_layout_passes=False),
        scratch_types=[
            pltpu.VMEM((BB * L,), jnp.int32),
            pltpu.VMEM((NP, 2 * BB), jnp.int32),
            pltpu.VMEM((NP, 2 * BB), jnp.int32),
            pltpu.VMEM((NP, 2 * BB), jnp.int32),
            pltpu.VMEM((2 * BB, D), jnp.float32),
            pltpu.VMEM((2 * BB, D), jnp.float32),
            pltpu.VMEM((2 * BB, D), jnp.float32),
            pltpu.VMEM((2 * BB, D), jnp.float32),
            pltpu.VMEM((2 * BB, D), jnp.float32),
            pltpu.VMEM((2 * BB, D), jnp.float32),
            pltpu.VMEM((2, OUTD // 8, 8, BB), jnp.float32),
            pltpu.VMEM((2, OUTD // 8, 8, BB), jnp.float32),
            pltpu.SemaphoreType.DMA,
            pltpu.SemaphoreType.DMA,
        ],
    )
    def k(idx0_h, idx1_h, idx2_h, emb0_h, emb1_h, emb2_h, out_h,
          iv, ivT0, ivT1, ivT2,
          g00, g01, g02, g10, g11, g12, t0, t1, gsem, wsem):
        wid = lax.axis_index("s") * nc + lax.axis_index("c")
        base = wid * (BB * L)

        ivTs = (ivT0, ivT1, ivT2)
        embs = (emb0_h, emb1_h, emb2_h)
        gbufs = ((g00, g01, g02), (g10, g11, g12))
        tbufs = (t0, t1)

        lane = lax.broadcasted_iota(jnp.int32, (16,), 0)
        laneL = lane * L

        for f in range(3):
            pltpu.sync_copy((idx0_h, idx1_h, idx2_h)[f].at[pl.ds(base, BB * L)],
                            iv)

            def idx_t_body(l, carry, f=f):
                p = l // 2
                sub = l - p * 2
                for g in range(BB // 16):
                    v = plsc.load_gather(iv, [laneL + (g * 16 * L + l)])
                    ivTs[f][p, pl.ds(sub * BB + g * 16, 16)] = v
                return carry

            lax.fori_loop(0, L, idx_t_body, 0)

        def fire_gather(p, par):
            for f in range(3):
                pltpu.async_copy(embs[f].at[ivTs[f].at[p]],
                                 gbufs[par][f], gsem)

        def wait_gather(par):
            for f in range(3):
                pltpu.make_async_copy(embs[f].at[ivTs[f].at[0]],
                                      gbufs[par][f], gsem).wait()

        def transpose(par):
            def tr_body(d, carry):
                r = d // 8
                s = d - r * 8
                cols = jnp.zeros((16,), jnp.int32) + d
                for f in range(3):
                    for sub in range(2):
                        for g in range(BB // 16):
                            rows = (sub * BB + g * 16) + lane
                            v = plsc.load_gather(gbufs[par][f], [rows, cols])
                            tbufs[par][sub, f * 4 + r, s,
                                       pl.ds(g * 16, 16)] = v
                return carry

            lax.fori_loop(0, D, tr_body, 0)

        def fire_write(p, par):
            pltpu.async_copy(tbufs[par], out_h.at[pl.ds(2 * p, 2), :, wid],
                             wsem)

        def wait_write(par):
            pltpu.make_async_copy(tbufs[par], out_h.at[pl.ds(0, 2), :, wid],
                                  wsem).wait()

        fire_gather(0, 0)
        fire_gather(1, 1)
        wait_gather(0)
        transpose(0)
        fire_write(0, 0)
        fire_gather(2, 0)
        wait_gather(1)
        transpose(1)
        fire_write(1, 1)

        def body(gi, carry):
            for bpar in range(2):
                p = 2 + gi * 2 + bpar
                par = bpar
                fire_gather(p + 1, 1 - par)
                wait_write(par)
                wait_gather(par)
                transpose(par)
                fire_write(p, par)
            return carry

        lax.fori_loop(0, (NP - 3) // 2, body, 0)

        wait_write(0)
        wait_gather(0)
        transpose(0)
        fire_write(NP - 1, 0)
        wait_write(1)
        wait_write(0)

    return k


_kern = _make_kernel()


def kernel(idx0, idx1, idx2, emb0, emb1, emb2):
    out5 = _kern(idx0.reshape(N), idx1.reshape(N), idx2.reshape(N),
                 emb0, emb1, emb2)
    return jnp.transpose(out5, (2, 4, 0, 1, 3)).reshape(B, L, OUTD)


# speedup vs baseline: 1.0016x; 1.0016x over previous
import functools

import jax
import jax.numpy as jnp
from jax import lax
from jax.experimental import pallas as pl
from jax.experimental.pallas import tpu as pltpu
from jax.experimental.pallas import tpu_sc as plsc

B, L = 4096, 50
N = B * L
D = 32
OUTD = 3 * D
NW = 32
BB = B // NW
NP = L // 2


def _make_kernel():
    info = plsc.get_sparse_core_info()
    nc = info.num_cores
    mesh = plsc.VectorSubcoreMesh(core_axis_name="c", subcore_axis_name="s")

    @functools.partial(
        pl.kernel,
        mesh=mesh,
        out_type=jax.ShapeDtypeStruct((L, OUTD // 8, NW, 8, BB), jnp.float32),
        compiler_params=pltpu.CompilerParams(use_tc_tiling_on_sc=False,
                                             needs_layout_passes=False),
        scratch_types=[
            pltpu.VMEM((BB * L,), jnp.int32),
            pltpu.VMEM((NP, 2 * BB), jnp.int32),
            pltpu.VMEM((NP, 2 * BB), jnp.int32),
            pltpu.VMEM((NP, 2 * BB), jnp.int32),
            pltpu.VMEM((2 * BB, D), jnp.float32),
            pltpu.VMEM((2 * BB, D), jnp.float32),
            pltpu.VMEM((2 * BB, D), jnp.float32),
            pltpu.VMEM((2 * BB, D), jnp.float32),
            pltpu.VMEM((2 * BB, D), jnp.float32),
            pltpu.VMEM((2 * BB, D), jnp.float32),
            pltpu.VMEM((2, OUTD // 8, 8, BB), jnp.float32),
            pltpu.VMEM((2, OUTD // 8, 8, BB), jnp.float32),
            pltpu.SemaphoreType.DMA,
            pltpu.SemaphoreType.DMA,
        ],
    )
    def k(idx0_h, idx1_h, idx2_h, emb0_h, emb1_h, emb2_h, out_h,
          iv, ivT0, ivT1, ivT2,
          g00, g01, g02, g10, g11, g12, t0, t1, gsem, wsem):
        wid = lax.axis_index("s") * nc + lax.axis_index("c")
        base = wid * (BB * L)

        ivTs = (ivT0, ivT1, ivT2)
        embs = (emb0_h, emb1_h, emb2_h)
        gbufs = ((g00, g01, g02), (g10, g11, g12))
        tbufs = (t0, t1)

        lane = lax.broadcasted_iota(jnp.int32, (16,), 0)
        laneL = lane * L

        for f in range(3):
            pltpu.sync_copy((idx0_h, idx1_h, idx2_h)[f].at[pl.ds(base, BB * L)],
                            iv)

            def idx_t_body(l, carry, f=f):
                p = l // 2
                sub = l - p * 2
                for g in range(BB // 16):
                    v = plsc.load_gather(iv, [laneL + (g * 16 * L + l)])
                    ivTs[f][p, pl.ds(sub * BB + g * 16, 16)] = v
                return carry

            lax.fori_loop(0, L, idx_t_body, 0)

        def fire_gather(p, par):
            for f in range(3):
                pltpu.async_copy(embs[f].at[ivTs[f].at[p]],
                                 gbufs[par][f], gsem)

        def wait_gather(par):
            for f in range(3):
                pltpu.make_async_copy(embs[f].at[ivTs[f].at[0]],
                                      gbufs[par][f], gsem).wait()

        def transpose(par):
            def tr_body(g, carry):
                for f in range(3):
                    for sub in range(2):
                        rows = (sub * BB + g * 16) + lane
                        for d in range(D):
                            cols = jnp.zeros((16,), jnp.int32) + d
                            v = plsc.load_gather(gbufs[par][f], [rows, cols])
                            tbufs[par][sub, f * 4 + d // 8, d % 8,
                                       pl.ds(g * 16, 16)] = v
                return carry

            lax.fori_loop(0, BB // 16, tr_body, 0)

        def fire_write(p, par):
            pltpu.async_copy(tbufs[par], out_h.at[pl.ds(2 * p, 2), :, wid],
                             wsem)

        def wait_write(par):
            pltpu.make_async_copy(tbufs[par], out_h.at[pl.ds(0, 2), :, wid],
                                  wsem).wait()

        fire_gather(0, 0)

        def body(gi, carry):
            for par in range(2):
                p = gi * 2 + par

                @pl.when(p < NP - 1)
                def _():
                    fire_gather(p + 1, 1 - par)

                @pl.when(jnp.logical_and(p >= 2, p < NP))
                def _():
                    wait_write(par)

                @pl.when(p < NP)
                def _():
                    wait_gather(par)
                    transpose(par)
                    fire_write(p, par)

            return carry

        lax.fori_loop(0, (NP + 2) // 2, body, 0)
        wait_write(0)
        wait_write(1)

    return k


_kern = _make_kernel()


def kernel(idx0, idx1, idx2, emb0, emb1, emb2):
    out5 = _kern(idx0.reshape(N), idx1.reshape(N), idx2.reshape(N),
                 emb0, emb1, emb2)
    return jnp.transpose(out5, (2, 4, 0, 1, 3)).reshape(B, L, OUTD)


# speedup vs baseline: 1.4508x; 1.4484x over previous
import functools

import jax
import jax.numpy as jnp
from jax import lax
from jax.experimental import pallas as pl
from jax.experimental.pallas import tpu as pltpu
from jax.experimental.pallas import tpu_sc as plsc

B, L = 4096, 50
N = B * L
D = 32
DP = D + 1
OUTD = 3 * D
NW = 32
BB = B // NW
NP = L // 2


def _make_kernel():
    info = plsc.get_sparse_core_info()
    nc = info.num_cores
    mesh = plsc.VectorSubcoreMesh(core_axis_name="c", subcore_axis_name="s")

    @functools.partial(
        pl.kernel,
        mesh=mesh,
        out_type=jax.ShapeDtypeStruct((L, OUTD // 8, NW, 8, BB), jnp.float32),
        compiler_params=pltpu.CompilerParams(use_tc_tiling_on_sc=False,
                                             needs_layout_passes=False),
        scratch_types=[
            pltpu.VMEM((BB * L,), jnp.int32),
            pltpu.VMEM((NP, 2 * BB), jnp.int32),
            pltpu.VMEM((NP, 2 * BB), jnp.int32),
            pltpu.VMEM((NP, 2 * BB), jnp.int32),
            pltpu.VMEM((2 * BB, D), jnp.float32),
            pltpu.VMEM((2 * BB, D), jnp.float32),
            pltpu.VMEM((2 * BB, D), jnp.float32),
            pltpu.VMEM((2 * BB, D), jnp.float32),
            pltpu.VMEM((2 * BB, D), jnp.float32),
            pltpu.VMEM((2 * BB, D), jnp.float32),
            pltpu.VMEM((OUTD // 8, 8, BB), jnp.float32),
            pltpu.VMEM((OUTD // 8, 8, BB), jnp.float32),
            pltpu.SemaphoreType.DMA,
            pltpu.SemaphoreType.DMA,
        ],
    )
    def k(idx0_h, idx1_h, idx2_h, emb0_h, emb1_h, emb2_h, out_h,
          iv, ivT0, ivT1, ivT2,
          g00, g01, g02, g10, g11, g12, ta, tb,
          gsem, wsem):
        wid = lax.axis_index("s") * nc + lax.axis_index("c")
        base = wid * (BB * L)

        ivTs = (ivT0, ivT1, ivT2)
        embs = (emb0_h, emb1_h, emb2_h)
        gbufs = ((g00, g01, g02), (g10, g11, g12))
        tbufs = (ta, tb)

        lane = lax.broadcasted_iota(jnp.int32, (16,), 0)
        laneL = lane * L

        for f in range(3):
            pltpu.sync_copy((idx0_h, idx1_h, idx2_h)[f].at[pl.ds(base, BB * L)],
                            iv)

            def idx_t_body(l, carry, f=f):
                p = l // 2
                sub = l - p * 2
                for g in range(BB // 16):
                    v = plsc.load_gather(iv, [laneL + (g * 16 * L + l)])
                    ivTs[f][p, pl.ds(sub * BB + g * 16, 16)] = v
                return carry

            lax.fori_loop(0, L, idx_t_body, 0)

        def fire_gather(p, par):
            for f in range(3):
                pltpu.async_copy(embs[f].at[ivTs[f].at[p]],
                                 gbufs[par][f], gsem)

        def wait_gather(par):
            for f in range(3):
                pltpu.make_async_copy(embs[f].at[ivTs[f].at[0]],
                                      gbufs[par][f], gsem).wait()

        def transpose_sub(par, sub):
            def tr_body(d0, carry):
                dvec = jnp.bitwise_and(d0 + lane, D - 1)
                svec = jnp.bitwise_and(dvec, 7)
                fvec = lax.shift_right_logical(dvec, 3)
                for f in range(3):
                    fiv = fvec + (f * 4)
                    for g in range(BB // 16):
                        rows = (sub * BB + g * 16) + lane
                        bvec = g * 16 + lane
                        v = plsc.load_gather(gbufs[par][f], [rows, dvec])
                        plsc.store_scatter(tbufs[sub], [fiv, svec, bvec], v)
                return carry

            lax.fori_loop(0, D, tr_body, 0)

        def fire_write(p, par, sub):
            pltpu.async_copy(tbufs[sub], out_h.at[2 * p + sub, :, wid], wsem)

        def wait_write(sub):
            pltpu.make_async_copy(tbufs[sub], out_h.at[0, :, wid],
                                  wsem).wait()

        fire_gather(0, 0)

        def body(gi, carry):
            for par in range(2):
                p = gi * 2 + par

                @pl.when(p < NP - 1)
                def _():
                    fire_gather(p + 1, 1 - par)

                @pl.when(p < NP)
                def _():
                    wait_gather(par)
                    for sub in range(2):
                        @pl.when(p >= 1)
                        def _():
                            wait_write(sub)

                        transpose_sub(par, sub)
                        fire_write(p, par, sub)

            return carry

        lax.fori_loop(0, (NP + 2) // 2, body, 0)
        wait_write(0)
        wait_write(1)

    return k


_kern = _make_kernel()


def kernel(idx0, idx1, idx2, emb0, emb1, emb2):
    out5 = _kern(idx0.reshape(N), idx1.reshape(N), idx2.reshape(N),
                 emb0, emb1, emb2)
    return jnp.transpose(out5, (2, 4, 0, 1, 3)).reshape(B, L, OUTD)
